# baseline (speedup 1.0000x reference)
import dataclasses

import jax
import jax.numpy as jnp
from jax import lax
from jax.experimental import pallas as pl
from jax.experimental.pallas import tpu as pltpu
from jax.experimental.pallas import tpu_sc as plsc

_B = 16384
_D = 32
_NC = 2
_NS = 16
_NW = _NC * _NS
_BPW = _B // _NW
_CH = 128
_NCH = _BPW // _CH
_L = 16


def _softplus(x):
    u = jnp.exp(-jnp.abs(x))
    z = u / (2.0 + u)
    z2 = z * z
    p = 2.0 * z * (1.0 + z2 * (1.0 / 3.0 + z2 * (0.2 + z2 * (1.0 / 7.0 + z2 * (1.0 / 9.0)))))
    return jnp.maximum(x, 0.0) + p


def _mirt_body(uid_hbm, qid_hbm, th_hbm, a_hbm, b_hbm, out_hbm,
               uid_v, qid_v, ti_v, ai_v, bi_v,
               th_v0, th_v1, a_v0, a_v1, b_v0, b_v1, out_v, sem):
    wid = lax.axis_index("s") * _NC + lax.axis_index("c")
    base = wid * _BPW

    idx_copies = []
    for j in range(_NCH):
        sl = pl.ds(base + j * _CH, _CH)
        idx_copies.append(pltpu.async_copy(uid_hbm.at[sl], uid_v.at[j], sem))
        idx_copies.append(pltpu.async_copy(qid_hbm.at[sl], qid_v.at[j], sem))
    for c in idx_copies:
        c.wait()

    for j in range(_NCH):
        for k in range(_CH // _L):
            s = pl.ds(k * _L, _L)
            uv = uid_v[j, s]
            qv = qid_v[j, s]
            ti_v[j, s] = lax.shift_right_logical(uv, 2)
            ai_v[j, s] = lax.shift_right_logical(qv, 2)
            bi_v[j, s] = lax.shift_right_logical(qv, 7)

    bufs = [(th_v0, a_v0, b_v0), (th_v1, a_v1, b_v1)]
    gathers = [None] * _NCH

    def issue(c):
        tb, ab, bb = bufs[c % 2]
        gathers[c] = [
            pltpu.async_copy(th_hbm.at[ti_v.at[c]], tb, sem),
            pltpu.async_copy(a_hbm.at[ai_v.at[c]], ab, sem),
            pltpu.async_copy(b_hbm.at[bi_v.at[c]], bb, sem),
        ]

    issue(0)
    lane = lax.iota(jnp.int32, _L)
    for c in range(_NCH):
        if c + 1 < _NCH:
            issue(c + 1)
        for g in gathers[c]:
            g.wait()
        tb, ab, bb = bufs[c % 2]

        @pl.loop(0, _CH, step=_L)
        def _(r0, c=c, tb=tb, ab=ab, bb=bb):
            rows = r0 + lane
            uv = uid_v[c, pl.ds(r0, _L)]
            qv = qid_v[c, pl.ds(r0, _L)]
            off_t = lax.shift_left(lax.bitwise_and(uv, 3), 5)
            off_a = lax.shift_left(lax.bitwise_and(qv, 3), 5)
            acc = None
            for d in range(_D):
                tv = plsc.load_gather(tb, [rows, off_t + d])
                av = plsc.load_gather(ab, [rows, off_a + d])
                term = _softplus(av) * tv
                acc = term if acc is None else acc + term
            bv = plsc.load_gather(bb, [rows, lax.bitwise_and(qv, _CH - 1)])
            zz = acc - bv
            out_v[pl.ds(c * _CH + r0, _L)] = 1.0 / (1.0 + jnp.exp(-zz))

    pltpu.sync_copy(out_v, out_hbm.at[pl.ds(base, _BPW)])


def kernel(user_id, question_id, theta_table, a_table, b_table):
    mesh = plsc.VectorSubcoreMesh(core_axis_name="c", subcore_axis_name="s")
    cp = pltpu.CompilerParams()
    if "needs_layout_passes" in pltpu.CompilerParams.__dataclass_fields__:
        cp = dataclasses.replace(cp, needs_layout_passes=False)
    run = pl.kernel(
        _mirt_body,
        out_type=jax.ShapeDtypeStruct((_B,), jnp.float32),
        mesh=mesh,
        scratch_types=[
            pltpu.VMEM((_NCH, _CH), jnp.int32),
            pltpu.VMEM((_NCH, _CH), jnp.int32),
            pltpu.VMEM((_NCH, _CH), jnp.int32),
            pltpu.VMEM((_NCH, _CH), jnp.int32),
            pltpu.VMEM((_NCH, _CH), jnp.int32),
            pltpu.VMEM((_CH, _CH), jnp.float32),
            pltpu.VMEM((_CH, _CH), jnp.float32),
            pltpu.VMEM((_CH, _CH), jnp.float32),
            pltpu.VMEM((_CH, _CH), jnp.float32),
            pltpu.VMEM((_CH, _CH), jnp.float32),
            pltpu.VMEM((_CH, _CH), jnp.float32),
            pltpu.VMEM((_BPW,), jnp.float32),
            pltpu.SemaphoreType.DMA,
        ],
        compiler_params=cp,
    )
    nq = a_table.shape[0]
    th128 = theta_table.reshape(theta_table.shape[0] * _D // _CH, _CH)
    a128 = a_table.reshape(nq * _D // _CH, _CH)
    bpad = jnp.concatenate([b_table.reshape(-1),
                            jnp.zeros((-nq) % _CH, jnp.float32)])
    b128 = bpad.reshape((nq + _CH - 1) // _CH, _CH)
    return run(user_id.astype(jnp.int32), question_id.astype(jnp.int32),
               th128, a128, b128)
